# baseline (speedup 1.0000x reference)
import functools

import jax
import jax.numpy as jnp
from jax import lax
from jax.experimental import pallas as pl
from jax.experimental.pallas import tpu as pltpu
from jax.experimental.pallas import tpu_sc as plsc

_B, _C, _H, _W, _K = 16, 2, 128, 128, 128
_HW = _H * _W
_L = 16
_PAD = 128


def _loss_body(outflat, combo, out,
               cmb_v, idx0_v, idx1_v, pred0_v, pred1_v,
               part2_v, gath_v, out_v, shared, sem_c, sem_g):
    cid = lax.axis_index("c")
    sid = lax.axis_index("s")
    b = sid

    @pl.when(cid == 0)
    def _core0():
        _tile_work(outflat, combo, out, b, sid,
                   cmb_v, idx0_v, idx1_v, pred0_v, pred1_v,
                   part2_v, gath_v, out_v, shared, sem_c, sem_g)


def _tile_work(outflat, combo, out, b, sid,
               cmb_v, idx0_v, idx1_v, pred0_v, pred1_v,
               part2_v, gath_v, out_v, shared, sem_c, sem_g):
    pltpu.sync_copy(combo.at[b], cmb_v)

    base0 = (2 * b) * _HW
    iota = lax.broadcasted_iota(jnp.int32, (_L,), 0)
    kidx0 = iota // 2
    choff = (iota % 2) * _HW
    for j in range(_K * _C // _L):
        vals_f = plsc.load_gather(cmb_v, [512 + 8 * j + kidx0])
        vals = plsc.bitcast(vals_f, jnp.int32)
        chunk = vals + (choff + base0)
        if j < 8:
            idx0_v[pl.ds(j * _L, _L)] = chunk
        else:
            idx1_v[pl.ds((j - 8) * _L, _L)] = chunk

    d0 = pltpu.async_copy(outflat.at[idx0_v], pred0_v, sem_g)
    d1 = pltpu.async_copy(outflat.at[idx1_v], pred1_v, sem_g)
    d0.wait()
    d1.wait()

    accl = jnp.zeros((_L,), jnp.float32)
    accm = jnp.zeros((_L,), jnp.float32)
    for i in range(_K * _C // _L):
        p = (pred0_v if i < 8 else pred1_v)[pl.ds((i % 8) * _L, _L)]
        m = cmb_v[pl.ds(i * _L, _L)]
        t = cmb_v[pl.ds(256 + i * _L, _L)]
        accl = accl + jnp.abs(p * m - t * m)
        accm = accm + m

    part2_v[0, :] = accl
    part2_v[1, :] = accm
    pltpu.sync_copy(part2_v, shared.at[pl.ds(_PAD + 2 * b, 2)])
    plsc.subcore_barrier()

    @pl.when(sid == 0)
    def _finalize():
        pltpu.sync_copy(shared.at[pl.ds(_PAD, 2 * _B)], gath_v)
        suml = jnp.zeros((_L,), jnp.float32)
        summ = jnp.zeros((_L,), jnp.float32)
        for i in range(_B):
            suml = suml + gath_v[2 * i, :]
            summ = summ + gath_v[2 * i + 1, :]
        sl = jnp.sum(suml)
        sm = jnp.sum(summ)
        num = jnp.full((_L,), sl, jnp.float32)
        den = jnp.full((_L,), sm, jnp.float32) + jnp.float32(1e-4)
        out_v[...] = num / den
        pltpu.sync_copy(out_v, out)


_sc_loss = functools.partial(
    pl.kernel,
    mesh=plsc.VectorSubcoreMesh(core_axis_name="c", subcore_axis_name="s"),
    compiler_params=pltpu.CompilerParams(needs_layout_passes=False),
    out_type=jax.ShapeDtypeStruct((_L,), jnp.float32),
    scratch_types=[
        pltpu.VMEM((2 * _K * _C + _K,), jnp.float32),
        pltpu.VMEM((_K * _C // 2,), jnp.int32),
        pltpu.VMEM((_K * _C // 2,), jnp.int32),
        pltpu.VMEM((_K * _C // 2,), jnp.float32),
        pltpu.VMEM((_K * _C // 2,), jnp.float32),
        pltpu.VMEM((2, _L), jnp.float32),
        pltpu.VMEM((2 * _B, _L), jnp.float32),
        pltpu.VMEM((_L,), jnp.float32),
        pltpu.VMEM_SHARED((_PAD + 2 * _B, _L), jnp.float32),
        pltpu.SemaphoreType.DMA,
        pltpu.SemaphoreType.DMA,
    ],
)(_loss_body)


def kernel(output, mask, ind, target):
    B, C, H, W = output.shape
    K = ind.shape[1]
    assert (B, C, H, W, K) == (_B, _C, _H, _W, _K)
    outflat = output.reshape(B * C * H * W)
    indf = jax.lax.bitcast_convert_type(ind, jnp.float32)
    combo = jnp.concatenate(
        [mask.reshape(B, K * C), target.reshape(B, K * C), indf], axis=1)
    res = _sc_loss(outflat, combo)
    return res[0]


# speedup vs baseline: 1.0007x; 1.0007x over previous
import functools

import jax
import jax.numpy as jnp
from jax import lax
from jax.experimental import pallas as pl
from jax.experimental.pallas import tpu as pltpu
from jax.experimental.pallas import tpu_sc as plsc

_B, _C, _H, _W, _K = 16, 2, 128, 128, 128
_HW = _H * _W
_L = 16
_PAD = 128


def _loss_body(outflat, combo, out,
               cmb_v, idx0_v, idx1_v, pred0_v, pred1_v,
               part2_v, gath_v, out_v, shared, sem_c, sem_g):
    cid = lax.axis_index("c")
    sid = lax.axis_index("s")
    b = sid

    @pl.when(cid == 0)
    def _core0():
        _tile_work(outflat, combo, out, b, sid,
                   cmb_v, idx0_v, idx1_v, pred0_v, pred1_v,
                   part2_v, gath_v, out_v, shared, sem_c, sem_g)


def _tile_work(outflat, combo, out, b, sid,
               cmb_v, idx0_v, idx1_v, pred0_v, pred1_v,
               part2_v, gath_v, out_v, shared, sem_c, sem_g):
    dc = pltpu.async_copy(combo.at[b], cmb_v, sem_c)
    dc.wait()

    base0 = (2 * b) * _HW
    iota = lax.broadcasted_iota(jnp.int32, (_L,), 0)
    kidx0 = iota // 2
    choff = (iota % 2) * _HW
    for j in range(_K * _C // _L):
        vals_f = plsc.load_gather(cmb_v, [512 + 8 * j + kidx0])
        vals = plsc.bitcast(vals_f, jnp.int32)
        chunk = vals + (choff + base0)
        if j < 8:
            idx0_v[pl.ds(j * _L, _L)] = chunk
        else:
            idx1_v[pl.ds((j - 8) * _L, _L)] = chunk

    d0 = pltpu.async_copy(outflat.at[idx0_v], pred0_v, sem_g)
    d1 = pltpu.async_copy(outflat.at[idx1_v], pred1_v, sem_g)

    accm = jnp.zeros((_L,), jnp.float32)
    for i in range(_K * _C // _L):
        accm = accm + cmb_v[pl.ds(i * _L, _L)]
    d0.wait()
    d1.wait()

    accl = jnp.zeros((_L,), jnp.float32)
    for i in range(_K * _C // _L):
        p = (pred0_v if i < 8 else pred1_v)[pl.ds((i % 8) * _L, _L)]
        m = cmb_v[pl.ds(i * _L, _L)]
        t = cmb_v[pl.ds(256 + i * _L, _L)]
        accl = accl + jnp.abs(p * m - t * m)

    part2_v[0, :] = accl
    part2_v[1, :] = accm
    pltpu.sync_copy(part2_v, shared.at[pl.ds(_PAD + 2 * b, 2)])
    plsc.subcore_barrier()

    @pl.when(sid == 0)
    def _finalize():
        pltpu.sync_copy(shared.at[pl.ds(_PAD, 2 * _B)], gath_v)
        suml = jnp.zeros((_L,), jnp.float32)
        summ = jnp.zeros((_L,), jnp.float32)
        for i in range(_B):
            suml = suml + gath_v[2 * i, :]
            summ = summ + gath_v[2 * i + 1, :]
        sl = jnp.sum(suml)
        sm = jnp.sum(summ)
        num = jnp.full((_L,), sl, jnp.float32)
        den = jnp.full((_L,), sm, jnp.float32) + jnp.float32(1e-4)
        out_v[...] = num / den
        pltpu.sync_copy(out_v, out)


_sc_loss = functools.partial(
    pl.kernel,
    mesh=plsc.VectorSubcoreMesh(core_axis_name="c", subcore_axis_name="s"),
    compiler_params=pltpu.CompilerParams(needs_layout_passes=False),
    out_type=jax.ShapeDtypeStruct((_L,), jnp.float32),
    scratch_types=[
        pltpu.VMEM((2 * _K * _C + _K,), jnp.float32),
        pltpu.VMEM((_K * _C // 2,), jnp.int32),
        pltpu.VMEM((_K * _C // 2,), jnp.int32),
        pltpu.VMEM((_K * _C // 2,), jnp.float32),
        pltpu.VMEM((_K * _C // 2,), jnp.float32),
        pltpu.VMEM((2, _L), jnp.float32),
        pltpu.VMEM((2 * _B, _L), jnp.float32),
        pltpu.VMEM((_L,), jnp.float32),
        pltpu.VMEM_SHARED((_PAD + 2 * _B, _L), jnp.float32),
        pltpu.SemaphoreType.DMA,
        pltpu.SemaphoreType.DMA,
    ],
)(_loss_body)


def kernel(output, mask, ind, target):
    B, C, H, W = output.shape
    K = ind.shape[1]
    assert (B, C, H, W, K) == (_B, _C, _H, _W, _K)
    outflat = output.reshape(B * C * H * W)
    indf = jax.lax.bitcast_convert_type(ind, jnp.float32)
    combo = jnp.concatenate(
        [mask.reshape(B, K * C), target.reshape(B, K * C), indf], axis=1)
    res = _sc_loss(outflat, combo)
    return res[0]


# speedup vs baseline: 1.0135x; 1.0128x over previous
import functools

import jax
import jax.numpy as jnp
from jax import lax
from jax.experimental import pallas as pl
from jax.experimental.pallas import tpu as pltpu
from jax.experimental.pallas import tpu_sc as plsc

_B, _C, _H, _W, _K = 16, 2, 128, 128, 128
_HW = _H * _W
_L = 16
_PAD = 128


def _loss_body(outflat, ind, maskf, targf, out,
               ind_v, idx0_v, idx1_v, pred0_v, pred1_v,
               mask_v, targ_v, part2_v, gath_v, out_v, shared,
               sem_i, sem_m, sem_t, sem_g):
    cid = lax.axis_index("c")
    sid = lax.axis_index("s")
    b = sid

    @pl.when(cid == 0)
    def _core0():
        _tile_work(outflat, ind, maskf, targf, out, b, sid,
                   ind_v, idx0_v, idx1_v, pred0_v, pred1_v,
                   mask_v, targ_v, part2_v, gath_v, out_v, shared,
                   sem_i, sem_m, sem_t, sem_g)


def _tile_work(outflat, ind, maskf, targf, out, b, sid,
               ind_v, idx0_v, idx1_v, pred0_v, pred1_v,
               mask_v, targ_v, part2_v, gath_v, out_v, shared,
               sem_i, sem_m, sem_t, sem_g):
    di = pltpu.async_copy(ind.at[b], ind_v, sem_i)
    dm = pltpu.async_copy(maskf.at[b], mask_v, sem_m)
    dt = pltpu.async_copy(targf.at[b], targ_v, sem_t)
    di.wait()

    base0 = (2 * b) * _HW
    iota = lax.broadcasted_iota(jnp.int32, (_L,), 0)
    kidx0 = iota // 2
    choff = (iota % 2) * _HW
    for j in range(8):
        vals = plsc.load_gather(ind_v, [8 * j + kidx0])
        idx0_v[pl.ds(j * _L, _L)] = vals + (choff + base0)
    d0 = pltpu.async_copy(outflat.at[idx0_v], pred0_v, sem_g)
    for j in range(8, 16):
        vals = plsc.load_gather(ind_v, [8 * j + kidx0])
        idx1_v[pl.ds((j - 8) * _L, _L)] = vals + (choff + base0)
    d1 = pltpu.async_copy(outflat.at[idx1_v], pred1_v, sem_g)

    dm.wait()
    accm = jnp.zeros((_L,), jnp.float32)
    for i in range(_K * _C // _L):
        accm = accm + mask_v[pl.ds(i * _L, _L)]
    dt.wait()
    d0.wait()
    d1.wait()

    accl = jnp.zeros((_L,), jnp.float32)
    for i in range(_K * _C // _L):
        p = (pred0_v if i < 8 else pred1_v)[pl.ds((i % 8) * _L, _L)]
        m = mask_v[pl.ds(i * _L, _L)]
        t = targ_v[pl.ds(i * _L, _L)]
        accl = accl + jnp.abs(p * m - t * m)

    part2_v[0, :] = accl
    part2_v[1, :] = accm
    pltpu.sync_copy(part2_v, shared.at[pl.ds(_PAD + 2 * b, 2)])
    plsc.subcore_barrier()

    @pl.when(sid == 0)
    def _finalize():
        pltpu.sync_copy(shared.at[pl.ds(_PAD, 2 * _B)], gath_v)
        suml = jnp.zeros((_L,), jnp.float32)
        summ = jnp.zeros((_L,), jnp.float32)
        for i in range(_B):
            suml = suml + gath_v[2 * i, :]
            summ = summ + gath_v[2 * i + 1, :]
        sl = jnp.sum(suml)
        sm = jnp.sum(summ)
        num = jnp.full((_L,), sl, jnp.float32)
        den = jnp.full((_L,), sm, jnp.float32) + jnp.float32(1e-4)
        out_v[...] = num / den
        pltpu.sync_copy(out_v, out)


_sc_loss = functools.partial(
    pl.kernel,
    mesh=plsc.VectorSubcoreMesh(core_axis_name="c", subcore_axis_name="s"),
    compiler_params=pltpu.CompilerParams(needs_layout_passes=False),
    out_type=jax.ShapeDtypeStruct((_L,), jnp.float32),
    scratch_types=[
        pltpu.VMEM((_K,), jnp.int32),
        pltpu.VMEM((_K * _C // 2,), jnp.int32),
        pltpu.VMEM((_K * _C // 2,), jnp.int32),
        pltpu.VMEM((_K * _C // 2,), jnp.float32),
        pltpu.VMEM((_K * _C // 2,), jnp.float32),
        pltpu.VMEM((_K * _C,), jnp.float32),
        pltpu.VMEM((_K * _C,), jnp.float32),
        pltpu.VMEM((2, _L), jnp.float32),
        pltpu.VMEM((2 * _B, _L), jnp.float32),
        pltpu.VMEM((_L,), jnp.float32),
        pltpu.VMEM_SHARED((_PAD + 2 * _B, _L), jnp.float32),
        pltpu.SemaphoreType.DMA,
        pltpu.SemaphoreType.DMA,
        pltpu.SemaphoreType.DMA,
        pltpu.SemaphoreType.DMA,
    ],
)(_loss_body)


def kernel(output, mask, ind, target):
    B, C, H, W = output.shape
    K = ind.shape[1]
    assert (B, C, H, W, K) == (_B, _C, _H, _W, _K)
    outflat = output.reshape(B * C * H * W)
    maskf = mask.reshape(B, K * C)
    targf = target.reshape(B, K * C)
    res = _sc_loss(outflat, ind, maskf, targf)
    return res[0]
